# baseline (speedup 1.0000x reference)
import functools

import jax
import jax.numpy as jnp
from jax import lax
from jax.experimental import pallas as pl
from jax.experimental.pallas import tpu as pltpu
from jax.experimental.pallas import tpu_sc as plsc

B = 4096
E = 200
EP = 256
H = 50
L = 16
NC, NS = 2, 16
NW = NC * NS
BPW = B // NW
NBLK = 13
NP = 100096
NBIAS = NP // 128
TCOL = 2176
TGRID = NP // TCOL
GM = 3.5


def _splat1(ref, i):
    return plsc.load_gather(ref, [jnp.full((L,), i, jnp.int32)])


def _linear_table(x):
    xt = x.T

    def body(xa_ref, xb_ref, o_ref):
        bt = xb_ref[...].T
        col = lax.broadcasted_iota(jnp.int32, (TCOL, 128), 1)
        bt = jnp.where(col < E - 128, bt, 0.0)
        rows = jnp.concatenate([xa_ref[...].T, bt], axis=1)
        o_ref[...] = rows.reshape(TCOL * EP)

    return pl.pallas_call(
        body,
        grid=(TGRID,),
        in_specs=[
            pl.BlockSpec((128, TCOL), lambda i: (0, i)),
            pl.BlockSpec((128, TCOL), lambda i: (1, i)),
        ],
        out_specs=pl.BlockSpec((TCOL * EP,), lambda i: (i,)),
        out_shape=jax.ShapeDtypeStruct((NP * EP,), jnp.float32),
    )(xt, xt)


def _svdpp_sc(user, movie, mr1, sq, iku, ikm, Bu2, Bi2, Pt, Qt, Yt):
    mesh = plsc.VectorSubcoreMesh(core_axis_name="c", subcore_axis_name="s")
    cp = pltpu.CompilerParams(
        needs_layout_passes=False, use_tc_tiling_on_sc=False
    )

    @functools.partial(
        pl.kernel,
        out_type=jax.ShapeDtypeStruct((B,), jnp.float32),
        mesh=mesh,
        compiler_params=cp,
        scratch_types=[
            pltpu.VMEM((BPW,), jnp.int32),
            pltpu.VMEM((BPW,), jnp.int32),
            pltpu.VMEM((BPW, H), jnp.int32),
            pltpu.VMEM((BPW,), jnp.float32),
            pltpu.VMEM((BPW,), jnp.float32),
            pltpu.VMEM((BPW,), jnp.float32),
            pltpu.VMEM((BPW,), jnp.int32),
            pltpu.VMEM((BPW, 128), jnp.float32),
            pltpu.VMEM((BPW,), jnp.float32),
            pltpu.VMEM((BPW,), jnp.float32),
            pltpu.VMEM((BPW, EP), jnp.float32),
            pltpu.VMEM((BPW, EP), jnp.float32),
            pltpu.VMEM((H, EP), jnp.float32),
            pltpu.VMEM((H, EP), jnp.float32),
            pltpu.VMEM((BPW,), jnp.float32),
            pltpu.SemaphoreType.DMA,
            pltpu.SemaphoreType.DMA,
            pltpu.SemaphoreType.DMA,
        ],
    )
    def kern(user_h, movie_h, mr_h, sq_h, iku_h, ikm_h, bu_h, bi_h, p_h,
             q_h, y_h, out_h, uidx, midx, mr_v, sq_v, iku_v, ikm_v, rowi,
             biasbuf, bu_val, bi_val, p_v, q_v, ybuf0, ybuf1, outv,
             sem_pre, sem0, sem1):
        wid = lax.axis_index("s") * NC + lax.axis_index("c")
        base = wid * BPW

        pltpu.sync_copy(user_h.at[pl.ds(base, BPW)], uidx)
        pltpu.sync_copy(movie_h.at[pl.ds(base, BPW)], midx)
        pltpu.sync_copy(mr_h.at[pl.ds(base, BPW)], mr_v)
        pltpu.sync_copy(sq_h.at[pl.ds(base, BPW)], sq_v)
        pltpu.sync_copy(iku_h.at[pl.ds(base, BPW)], iku_v)
        pltpu.sync_copy(ikm_h.at[pl.ds(base, BPW)], ikm_v)

        hp = pltpu.async_copy(p_h.at[uidx], p_v, sem_pre)
        hq = pltpu.async_copy(q_h.at[midx], q_v, sem_pre)

        pltpu.async_copy(y_h.at[mr_v.at[0]], ybuf0, sem0)

        lane = lax.iota(jnp.int32, L)
        tail_mask = lane < (E - 16 * (NBLK - 1))
        zero = jnp.zeros((L,), jnp.float32)

        def bias_values(idx_v, table_h, val_v):
            @pl.loop(0, BPW, step=L)
            def _(t):
                rowi[pl.ds(t, L)] = lax.shift_right_logical(
                    idx_v[pl.ds(t, L)], 7
                )

            pltpu.async_copy(table_h.at[rowi], biasbuf, sem_pre).wait()

            @pl.loop(0, BPW, step=L)
            def _(t):
                lanes = jnp.bitwise_and(idx_v[pl.ds(t, L)], 127)
                val_v[pl.ds(t, L)] = plsc.load_gather(
                    biasbuf, [lane + t, lanes]
                )

        bias_values(uidx, bu_h, bu_val)
        bias_values(midx, bi_h, bi_val)

        hp.wait()
        hq.wait()

        def compute(b, ybuf):
            def row_body(h, accs):
                return tuple(
                    accs[j] + ybuf[h, pl.ds(16 * j, L)] for j in range(NBLK)
                )

            accs = lax.fori_loop(0, H, row_body, (zero,) * NBLK)

            iku_s = _splat1(iku_v, b)
            ikm_s = _splat1(ikm_v, b)
            sq_s = _splat1(sq_v, b)
            ysc = iku_s / sq_s
            tsum = zero
            for j in range(NBLK - 1):
                pj = p_v[b, pl.ds(16 * j, L)]
                qj = q_v[b, pl.ds(16 * j, L)]
                tsum = tsum + qj * (pj * iku_s + accs[j] * ysc)
            pj = p_v[b, pl.ds(16 * (NBLK - 1), L)]
            qj = q_v[b, pl.ds(16 * (NBLK - 1), L)]
            tt = qj * (pj * iku_s + accs[NBLK - 1] * ysc)
            tsum = tsum + jnp.where(tail_mask, tt, zero)
            dot = jnp.sum(tsum)
            bu_s = _splat1(bu_val, b)
            bi_s = _splat1(bi_val, b)
            r = ikm_s * jnp.full((L,), dot, jnp.float32) \
                + bi_s * ikm_s + bu_s * iku_s + GM
            plsc.store_scatter(
                outv, [jnp.full((L,), b, jnp.int32)], r, mask=(lane == 0)
            )

        @pl.loop(0, BPW, step=2)
        def _(g):
            pltpu.async_copy(y_h.at[mr_v.at[g + 1]], ybuf1, sem1)
            pltpu.make_async_copy(y_h.at[mr_v.at[g]], ybuf0, sem0).wait()
            compute(g, ybuf0)

            @pl.when(g + 2 < BPW)
            def _():
                pltpu.async_copy(y_h.at[mr_v.at[g + 2]], ybuf0, sem0)

            pltpu.make_async_copy(
                y_h.at[mr_v.at[g + 1]], ybuf1, sem1
            ).wait()
            compute(g + 1, ybuf1)

        pltpu.sync_copy(outv, out_h.at[pl.ds(base, BPW)])

    return kern(user, movie, mr1, sq, iku, ikm, Bu2, Bi2, Pt, Qt, Yt)


@jax.jit
def kernel(user, movie, movies_rated_by_this_user, users_who_rated_this_movie,
           sqrt_of_number_of_movies_rated_by_this_user,
           sqrt_of_number_of_users_who_rated_this_movie,
           is_known_user, is_known_movie, Bu, Bi, P, Q, Y):
    del users_who_rated_this_movie, sqrt_of_number_of_users_who_rated_this_movie
    sq = sqrt_of_number_of_movies_rated_by_this_user.reshape(B)
    iku = is_known_user.reshape(B)
    ikm = is_known_movie.reshape(B)

    Pt = _linear_table(P).reshape(NP, EP)
    Qt = _linear_table(Q).reshape(NP, EP)
    Yt = _linear_table(Y).reshape(NP, EP)

    mr1 = movies_rated_by_this_user.astype(jnp.int32)

    Bu2 = jnp.pad(Bu.reshape(-1), (0, NP - Bu.shape[0])).reshape(NBIAS, 128)
    Bi2 = jnp.pad(Bi.reshape(-1), (0, NP - Bi.shape[0])).reshape(NBIAS, 128)

    return _svdpp_sc(user.astype(jnp.int32), movie.astype(jnp.int32), mr1,
                     sq, iku, ikm, Bu2, Bi2, Pt, Qt, Yt)


# speedup vs baseline: 1.2120x; 1.2120x over previous
import functools

import jax
import jax.numpy as jnp
from jax import lax
from jax.experimental import pallas as pl
from jax.experimental.pallas import tpu as pltpu
from jax.experimental.pallas import tpu_sc as plsc

B = 4096
E = 200
EP = 256
H = 50
L = 16
NC, NS = 2, 16
NW = NC * NS
BPW = B // NW
NBLK = 13
NP = 100096
NBIAS = NP // 128
TCOL = 2176
TGRID = NP // TCOL
GM = 3.5

_MESH = plsc.VectorSubcoreMesh(core_axis_name="c", subcore_axis_name="s")
_CP = pltpu.CompilerParams(
    needs_layout_passes=False, use_tc_tiling_on_sc=False
)


def _splat1(ref, i):
    return plsc.load_gather(ref, [jnp.full((L,), i, jnp.int32)])


def _linear_table(x):
    xt = x.T

    def body(xa_ref, xb_ref, o_ref):
        bt = xb_ref[...].T
        col = lax.broadcasted_iota(jnp.int32, (TCOL, 128), 1)
        bt = jnp.where(col < E - 128, bt, 0.0)
        rows = jnp.concatenate([xa_ref[...].T, bt], axis=1)
        o_ref[...] = rows.reshape(TCOL * EP)

    return pl.pallas_call(
        body,
        grid=(TGRID,),
        in_specs=[
            pl.BlockSpec((128, TCOL), lambda i: (0, i)),
            pl.BlockSpec((128, TCOL), lambda i: (1, i)),
        ],
        out_specs=pl.BlockSpec((TCOL * EP,), lambda i: (i,)),
        out_shape=jax.ShapeDtypeStruct((NP * EP,), jnp.float32),
    )(xt, xt)


def _ysum_sc(mr1, Yt):
    @functools.partial(
        pl.kernel,
        out_type=jax.ShapeDtypeStruct((B, EP), jnp.float32),
        mesh=_MESH,
        compiler_params=_CP,
        scratch_types=[
            pltpu.VMEM((BPW, H), jnp.int32),
            pltpu.VMEM((H, EP), jnp.float32),
            pltpu.VMEM((H, EP), jnp.float32),
            pltpu.VMEM((BPW, EP), jnp.float32),
            pltpu.SemaphoreType.DMA,
            pltpu.SemaphoreType.DMA,
        ],
    )
    def kern(mr_h, y_h, out_h, mr_v, ybuf0, ybuf1, ysv, sem0, sem1):
        wid = lax.axis_index("s") * NC + lax.axis_index("c")
        base = wid * BPW

        pltpu.sync_copy(mr_h.at[pl.ds(base, BPW)], mr_v)
        pltpu.async_copy(y_h.at[mr_v.at[0]], ybuf0, sem0)
        zero = jnp.zeros((L,), jnp.float32)

        def compute(b, ybuf):
            def row_body(h, accs):
                return tuple(
                    accs[j] + ybuf[h, pl.ds(16 * j, L)] for j in range(NBLK)
                )

            accs = lax.fori_loop(0, H, row_body, (zero,) * NBLK)
            for j in range(NBLK):
                ysv[b, pl.ds(16 * j, L)] = accs[j]

        @pl.loop(0, BPW, step=2)
        def _(g):
            pltpu.async_copy(y_h.at[mr_v.at[g + 1]], ybuf1, sem1)
            pltpu.make_async_copy(y_h.at[mr_v.at[g]], ybuf0, sem0).wait()
            compute(g, ybuf0)

            @pl.when(g + 2 < BPW)
            def _():
                pltpu.async_copy(y_h.at[mr_v.at[g + 2]], ybuf0, sem0)

            pltpu.make_async_copy(
                y_h.at[mr_v.at[g + 1]], ybuf1, sem1
            ).wait()
            compute(g + 1, ybuf1)

        pltpu.sync_copy(ysv, out_h.at[pl.ds(base, BPW)])

    return kern(mr1, Yt)


def _combine_sc(user, movie, sq, iku, ikm, Bu2, Bi2, Pt, Qt, ysum):
    @functools.partial(
        pl.kernel,
        out_type=jax.ShapeDtypeStruct((B,), jnp.float32),
        mesh=_MESH,
        compiler_params=_CP,
        scratch_types=[
            pltpu.VMEM((BPW,), jnp.int32),
            pltpu.VMEM((BPW,), jnp.int32),
            pltpu.VMEM((BPW,), jnp.float32),
            pltpu.VMEM((BPW,), jnp.float32),
            pltpu.VMEM((BPW,), jnp.float32),
            pltpu.VMEM((BPW,), jnp.int32),
            pltpu.VMEM((BPW, 128), jnp.float32),
            pltpu.VMEM((BPW,), jnp.float32),
            pltpu.VMEM((BPW,), jnp.float32),
            pltpu.VMEM((BPW, EP), jnp.float32),
            pltpu.VMEM((BPW, EP), jnp.float32),
            pltpu.VMEM((BPW, EP), jnp.float32),
            pltpu.VMEM((BPW,), jnp.float32),
            pltpu.SemaphoreType.DMA,
        ],
    )
    def kern(user_h, movie_h, sq_h, iku_h, ikm_h, bu_h, bi_h, p_h, q_h,
             ys_h, out_h, uidx, midx, sq_v, iku_v, ikm_v, rowi, biasbuf,
             bu_val, bi_val, p_v, q_v, ys_v, outv, sem_pre):
        wid = lax.axis_index("s") * NC + lax.axis_index("c")
        base = wid * BPW

        pltpu.sync_copy(user_h.at[pl.ds(base, BPW)], uidx)
        pltpu.sync_copy(movie_h.at[pl.ds(base, BPW)], midx)
        pltpu.sync_copy(sq_h.at[pl.ds(base, BPW)], sq_v)
        pltpu.sync_copy(iku_h.at[pl.ds(base, BPW)], iku_v)
        pltpu.sync_copy(ikm_h.at[pl.ds(base, BPW)], ikm_v)

        hp = pltpu.async_copy(p_h.at[uidx], p_v, sem_pre)
        hq = pltpu.async_copy(q_h.at[midx], q_v, sem_pre)
        hy = pltpu.async_copy(ys_h.at[pl.ds(base, BPW)], ys_v, sem_pre)

        lane = lax.iota(jnp.int32, L)
        tail_mask = lane < (E - 16 * (NBLK - 1))
        zero = jnp.zeros((L,), jnp.float32)

        def bias_values(idx_v, table_h, val_v):
            @pl.loop(0, BPW, step=L)
            def _(t):
                rowi[pl.ds(t, L)] = lax.shift_right_logical(
                    idx_v[pl.ds(t, L)], 7
                )

            pltpu.async_copy(table_h.at[rowi], biasbuf, sem_pre).wait()

            @pl.loop(0, BPW, step=L)
            def _(t):
                lanes = jnp.bitwise_and(idx_v[pl.ds(t, L)], 127)
                val_v[pl.ds(t, L)] = plsc.load_gather(
                    biasbuf, [lane + t, lanes]
                )

        bias_values(uidx, bu_h, bu_val)
        bias_values(midx, bi_h, bi_val)

        hp.wait()
        hq.wait()
        hy.wait()

        @pl.loop(0, BPW)
        def _(b):
            iku_s = _splat1(iku_v, b)
            ikm_s = _splat1(ikm_v, b)
            sq_s = _splat1(sq_v, b)
            ysc = iku_s / sq_s
            tsum = zero
            for j in range(NBLK - 1):
                pj = p_v[b, pl.ds(16 * j, L)]
                qj = q_v[b, pl.ds(16 * j, L)]
                yj = ys_v[b, pl.ds(16 * j, L)]
                tsum = tsum + qj * (pj * iku_s + yj * ysc)
            pj = p_v[b, pl.ds(16 * (NBLK - 1), L)]
            qj = q_v[b, pl.ds(16 * (NBLK - 1), L)]
            yj = ys_v[b, pl.ds(16 * (NBLK - 1), L)]
            tt = qj * (pj * iku_s + yj * ysc)
            tsum = tsum + jnp.where(tail_mask, tt, zero)
            dot = jnp.sum(tsum)
            bu_s = _splat1(bu_val, b)
            bi_s = _splat1(bi_val, b)
            r = ikm_s * jnp.full((L,), dot, jnp.float32) \
                + bi_s * ikm_s + bu_s * iku_s + GM
            plsc.store_scatter(
                outv, [jnp.full((L,), b, jnp.int32)], r, mask=(lane == 0)
            )

        pltpu.sync_copy(outv, out_h.at[pl.ds(base, BPW)])

    return kern(user, movie, sq, iku, ikm, Bu2, Bi2, Pt, Qt, ysum)


@jax.jit
def kernel(user, movie, movies_rated_by_this_user, users_who_rated_this_movie,
           sqrt_of_number_of_movies_rated_by_this_user,
           sqrt_of_number_of_users_who_rated_this_movie,
           is_known_user, is_known_movie, Bu, Bi, P, Q, Y):
    del users_who_rated_this_movie, sqrt_of_number_of_users_who_rated_this_movie
    sq = sqrt_of_number_of_movies_rated_by_this_user.reshape(B)
    iku = is_known_user.reshape(B)
    ikm = is_known_movie.reshape(B)

    Yt = _linear_table(Y).reshape(NP, EP)
    ysum = _ysum_sc(movies_rated_by_this_user.astype(jnp.int32), Yt)
    Pt = _linear_table(P).reshape(NP, EP)
    Qt = _linear_table(Q).reshape(NP, EP)

    Bu2 = jnp.pad(Bu.reshape(-1), (0, NP - Bu.shape[0])).reshape(NBIAS, 128)
    Bi2 = jnp.pad(Bi.reshape(-1), (0, NP - Bi.shape[0])).reshape(NBIAS, 128)

    return _combine_sc(user.astype(jnp.int32), movie.astype(jnp.int32),
                       sq, iku, ikm, Bu2, Bi2, Pt, Qt, ysum)
